# baseline (speedup 1.0000x reference)
import functools

import jax
import jax.numpy as jnp
from jax import lax
from jax.experimental import pallas as pl
from jax.experimental.pallas import tpu as pltpu
from jax.experimental.pallas import tpu_sc as plsc

NC = 2
NS = 16
LANES = 16

CH = 40

ROW_BLK = 1000


def _fc1_body(x_ref, w_ref, b_ref, o_ref):
    acc = lax.dot_general(x_ref[...], w_ref[...], (((1,), (1,)), ((), ())),
                          preferred_element_type=jnp.float32)
    o_ref[...] = jnp.maximum(acc + b_ref[...], 0.0)


def _fc2_body(p_ref, w_ref, b_ref, o_ref):
    agg = p_ref[0] + p_ref[1]
    b = b_ref[...]
    g = lax.dot_general(agg, w_ref[...], (((1,), (1,)), ((), ())),
                        preferred_element_type=jnp.float32) + b
    o = lax.dot_general(g, w_ref[...], (((1,), (1,)), ((), ())),
                        preferred_element_type=jnp.float32) + b
    m = jnp.max(o, axis=1, keepdims=True)
    e = jnp.exp(o - m)
    o_ref[...] = e / jnp.sum(e, axis=1, keepdims=True)


NBUF = 3


def _make_sc_gather_scatter(n, e, d):
    assert e % (NC * NS) == 0
    epw = e // (NC * NS)
    assert epw % CH == 0
    n_chunks = epw // CH
    n_outer = (n_chunks + NBUF - 1) // NBUF
    wb = (n // NS) & ~7
    rem = n - NS * wb
    assert rem % 8 == 0

    mesh = plsc.VectorSubcoreMesh(core_axis_name="c", subcore_axis_name="s")

    @functools.partial(
        pl.kernel,
        mesh=mesh,
        out_type=jax.ShapeDtypeStruct((NC, n, d), jnp.float32),
        scratch_types=[
            pltpu.VMEM((n_chunks, CH), jnp.int32),
            [pltpu.VMEM((2, CH // 2), jnp.int32) for _ in range(NBUF)],
            [pltpu.VMEM((48,), jnp.float32) for _ in range(NBUF)],
            [pltpu.VMEM((CH, d), jnp.float32) for _ in range(NBUF)],
            pltpu.VMEM_SHARED((n, d), jnp.float32),
            [pltpu.SemaphoreType.DMA for _ in range(NBUF)],
            [pltpu.SemaphoreType.DMA for _ in range(NBUF)],
        ],
    )
    def sc_kernel(h_hbm, src_hbm, dst_hbm, w_hbm, out_hbm,
                  src_v, dst_ring, w_ring, rows, acc_sh, gsem, ssem):
        cid = lax.axis_index("c")
        sid = lax.axis_index("s")
        wid = cid * NS + sid
        base = wid * epw

        pltpu.sync_copy(src_hbm.at[wid], src_v)

        zeros16 = jnp.zeros((LANES,), jnp.float32)

        def zrow(i, _):
            for j in range(d // LANES):
                rows[0][i, pl.ds(j * LANES, LANES)] = zeros16
            return 0

        lax.fori_loop(0, CH, zrow, 0)
        for k in range(wb // CH):
            pltpu.sync_copy(rows[0], acc_sh.at[pl.ds(sid * wb + k * CH, CH)])
        zr = wb - (wb // CH) * CH
        if zr:
            pltpu.sync_copy(rows[0].at[pl.ds(0, zr)],
                            acc_sh.at[pl.ds(sid * wb + (wb // CH) * CH, zr)])

        @pl.when(sid == NS - 1)
        def _zero_tail():
            pltpu.sync_copy(rows[0].at[pl.ds(0, rem)],
                            acc_sh.at[pl.ds(NS * wb, rem)])

        plsc.subcore_barrier()

        for b in range(NBUF - 1):
            pltpu.async_copy(w_hbm.at[pl.ds(base + b * CH, CH)],
                             w_ring[b].at[pl.ds(0, CH)], gsem[b])
            pltpu.async_copy(dst_hbm.at[wid, b], dst_ring[b], gsem[b])
            pltpu.async_copy(h_hbm.at[src_v.at[b]], rows[b], gsem[b])

        def outer(g, _):
            for b in range(NBUF):
                ci = g * NBUF + b

                @pl.when(ci < n_chunks)
                def _step():
                    pltpu.make_async_copy(w_hbm.at[pl.ds(base + ci * CH, CH)],
                                          w_ring[b].at[pl.ds(0, CH)],
                                          gsem[b]).wait()
                    pltpu.make_async_copy(dst_hbm.at[wid, ci], dst_ring[b],
                                          gsem[b]).wait()
                    pltpu.make_async_copy(h_hbm.at[src_v.at[ci]], rows[b],
                                          gsem[b]).wait()

                    def scale_range(r0, r1):
                        for gg in range(r0 // LANES, (r1 - 1) // LANES + 1):
                            w16 = w_ring[b][pl.ds(gg * LANES, LANES)]
                            for r in range(max(r0, gg * LANES),
                                           min(r1, (gg + 1) * LANES)):
                                wv = jnp.full((LANES,), w16[r - gg * LANES],
                                              dtype=jnp.float32)
                                for j in range(d // LANES):
                                    sl = pl.ds(j * LANES, LANES)
                                    rows[b][r, sl] = rows[b][r, sl] * wv

                    hh = CH // 2
                    scale_range(0, hh)
                    pltpu.async_copy(rows[b].at[pl.ds(0, hh)],
                                     acc_sh.at[dst_ring[b].at[0]], ssem[b],
                                     add=True)
                    scale_range(hh, CH)
                    pltpu.async_copy(rows[b].at[pl.ds(hh, hh)],
                                     acc_sh.at[dst_ring[b].at[1]], ssem[b],
                                     add=True)

                    bn = (b + NBUF - 1) % NBUF
                    cf = ci + NBUF - 1

                    @pl.when(cf < n_chunks)
                    def _prefetch():
                        @pl.when(ci >= 1)
                        def _drain():
                            pltpu.make_async_copy(
                                rows[bn].at[pl.ds(0, CH // 2)],
                                acc_sh.at[dst_ring[bn].at[0]],
                                ssem[bn]).wait()
                            pltpu.make_async_copy(
                                rows[bn].at[pl.ds(CH // 2, CH // 2)],
                                acc_sh.at[dst_ring[bn].at[1]],
                                ssem[bn]).wait()
                        pltpu.async_copy(w_hbm.at[pl.ds(base + cf * CH, CH)],
                                         w_ring[bn].at[pl.ds(0, CH)], gsem[bn])
                        pltpu.async_copy(dst_hbm.at[wid, cf], dst_ring[bn],
                                         gsem[bn])
                        pltpu.async_copy(h_hbm.at[src_v.at[cf]], rows[bn],
                                         gsem[bn])
            return 0

        lax.fori_loop(0, n_outer, outer, 0)

        for b in range(NBUF):
            pltpu.make_async_copy(rows[b].at[pl.ds(0, CH // 2)],
                                  acc_sh.at[dst_ring[b].at[0]],
                                  ssem[b]).wait()
            pltpu.make_async_copy(rows[b].at[pl.ds(CH // 2, CH // 2)],
                                  acc_sh.at[dst_ring[b].at[1]],
                                  ssem[b]).wait()

        plsc.subcore_barrier()
        pltpu.sync_copy(acc_sh.at[pl.ds(sid * wb, wb)],
                        out_hbm.at[cid, pl.ds(sid * wb, wb)])

        @pl.when(sid == NS - 1)
        def _wb_tail():
            pltpu.sync_copy(acc_sh.at[pl.ds(NS * wb, rem)],
                            out_hbm.at[cid, pl.ds(NS * wb, rem)])

    return sc_kernel


def kernel(x, edge_index, edge_weight, W1, b1, W2, b2):
    n, d_in = x.shape
    e = edge_weight.shape[0]
    d_h = W1.shape[0]
    d_out = W2.shape[0]

    b1r = b1.reshape(1, d_h)
    b2r = b2.reshape(1, d_out)

    grid = n // ROW_BLK
    h = pl.pallas_call(
        _fc1_body,
        grid=(grid,),
        in_specs=[
            pl.BlockSpec((ROW_BLK, d_in), lambda i: (i, 0)),
            pl.BlockSpec((d_h, d_in), lambda i: (0, 0)),
            pl.BlockSpec((1, d_h), lambda i: (0, 0)),
        ],
        out_specs=pl.BlockSpec((ROW_BLK, d_h), lambda i: (i, 0)),
        out_shape=jax.ShapeDtypeStruct((n, d_h), jnp.float32),
    )(x, W1, b1r)

    nw = NC * NS
    epw = e // nw
    src = edge_index[1].reshape(nw, epw // CH, CH)
    dst = edge_index[0].reshape(nw, epw // CH, 2, CH // 2)
    w = edge_weight
    partials = _make_sc_gather_scatter(n, e, d_h)(h, src, dst, w)

    out = pl.pallas_call(
        _fc2_body,
        grid=(grid,),
        in_specs=[
            pl.BlockSpec((NC, ROW_BLK, d_h), lambda i: (0, i, 0)),
            pl.BlockSpec((d_out, d_h), lambda i: (0, 0)),
            pl.BlockSpec((1, d_out), lambda i: (0, 0)),
        ],
        out_specs=pl.BlockSpec((ROW_BLK, d_out), lambda i: (i, 0)),
        out_shape=jax.ShapeDtypeStruct((n, d_out), jnp.float32),
    )(partials, W2, b2r)
    return out


# speedup vs baseline: 1.0654x; 1.0654x over previous
import functools

import jax
import jax.numpy as jnp
from jax import lax
from jax.experimental import pallas as pl
from jax.experimental.pallas import tpu as pltpu
from jax.experimental.pallas import tpu_sc as plsc

NC = 2
NS = 16
LANES = 16

CH = 40

ROW_BLK = 1000


def _fc1_body(x_ref, w_ref, b_ref, o_ref):
    acc = lax.dot_general(x_ref[...], w_ref[...], (((1,), (1,)), ((), ())),
                          preferred_element_type=jnp.float32)
    o_ref[...] = jnp.maximum(acc + b_ref[...], 0.0)


def _fc2_body(p_ref, w_ref, b_ref, o_ref):
    agg = p_ref[0] + p_ref[1]
    b = b_ref[...]
    g = lax.dot_general(agg, w_ref[...], (((1,), (1,)), ((), ())),
                        preferred_element_type=jnp.float32) + b
    o = lax.dot_general(g, w_ref[...], (((1,), (1,)), ((), ())),
                        preferred_element_type=jnp.float32) + b
    m = jnp.max(o, axis=1, keepdims=True)
    e = jnp.exp(o - m)
    o_ref[...] = e / jnp.sum(e, axis=1, keepdims=True)


NBUF = 3


def _make_sc_gather_scatter(n, e, d):
    assert e % (NC * NS) == 0
    epw = e // (NC * NS)
    assert epw % CH == 0
    n_chunks = epw // CH
    n_outer = (n_chunks + NBUF - 1) // NBUF
    wb = (n // NS) & ~7
    rem = n - NS * wb
    assert rem % 8 == 0

    mesh = plsc.VectorSubcoreMesh(core_axis_name="c", subcore_axis_name="s")

    @functools.partial(
        pl.kernel,
        mesh=mesh,
        out_type=jax.ShapeDtypeStruct((NC, n, d), jnp.float32),
        scratch_types=[
            pltpu.VMEM((n_chunks, CH), jnp.int32),
            [pltpu.VMEM((CH,), jnp.int32) for _ in range(NBUF)],
            [pltpu.VMEM((48,), jnp.float32) for _ in range(NBUF)],
            [pltpu.VMEM((CH, d), jnp.float32) for _ in range(NBUF)],
            pltpu.VMEM_SHARED((n, d), jnp.float32),
            [pltpu.SemaphoreType.DMA for _ in range(NBUF)],
            [pltpu.SemaphoreType.DMA for _ in range(NBUF)],
        ],
    )
    def sc_kernel(h_hbm, src_hbm, dst_hbm, w_hbm, out_hbm,
                  src_v, dst_ring, w_ring, rows, acc_sh, gsem, ssem):
        cid = lax.axis_index("c")
        sid = lax.axis_index("s")
        wid = cid * NS + sid
        base = wid * epw

        pltpu.sync_copy(src_hbm.at[wid], src_v)

        zeros16 = jnp.zeros((LANES,), jnp.float32)

        def zrow(i, _):
            for j in range(d // LANES):
                rows[0][i, pl.ds(j * LANES, LANES)] = zeros16
            return 0

        lax.fori_loop(0, CH, zrow, 0)
        for k in range(wb // CH):
            pltpu.sync_copy(rows[0], acc_sh.at[pl.ds(sid * wb + k * CH, CH)])
        zr = wb - (wb // CH) * CH
        if zr:
            pltpu.sync_copy(rows[0].at[pl.ds(0, zr)],
                            acc_sh.at[pl.ds(sid * wb + (wb // CH) * CH, zr)])

        @pl.when(sid == NS - 1)
        def _zero_tail():
            pltpu.sync_copy(rows[0].at[pl.ds(0, rem)],
                            acc_sh.at[pl.ds(NS * wb, rem)])

        plsc.subcore_barrier()

        for b in range(NBUF - 1):
            pltpu.async_copy(w_hbm.at[pl.ds(base + b * CH, CH)],
                             w_ring[b].at[pl.ds(0, CH)], gsem[b])
            pltpu.async_copy(dst_hbm.at[wid, b], dst_ring[b], gsem[b])
            pltpu.async_copy(h_hbm.at[src_v.at[b]], rows[b], gsem[b])

        def outer(g, _):
            for b in range(NBUF):
                ci = g * NBUF + b

                @pl.when(ci < n_chunks)
                def _step():
                    pltpu.make_async_copy(w_hbm.at[pl.ds(base + ci * CH, CH)],
                                          w_ring[b].at[pl.ds(0, CH)],
                                          gsem[b]).wait()
                    pltpu.make_async_copy(dst_hbm.at[wid, ci], dst_ring[b],
                                          gsem[b]).wait()
                    pltpu.make_async_copy(h_hbm.at[src_v.at[ci]], rows[b],
                                          gsem[b]).wait()

                    def scale16(gg, nrow):
                        w16 = w_ring[b][pl.ds(gg * LANES, LANES)]
                        for i in range(nrow):
                            wv = jnp.full((LANES,), w16[i], dtype=jnp.float32)
                            r = gg * LANES + i
                            for j in range(d // LANES):
                                sl = pl.ds(j * LANES, LANES)
                                rows[b][r, sl] = rows[b][r, sl] * wv

                    for gg in range(CH // LANES):
                        scale16(gg, LANES)
                    if CH % LANES:
                        scale16(CH // LANES, CH % LANES)

                    pltpu.async_copy(rows[b], acc_sh.at[dst_ring[b]], ssem[b],
                                     add=True)

                    bn = (b + NBUF - 1) % NBUF
                    cf = ci + NBUF - 1

                    @pl.when(cf < n_chunks)
                    def _prefetch():
                        @pl.when(ci >= 1)
                        def _drain():
                            pltpu.make_async_copy(
                                rows[bn], acc_sh.at[dst_ring[bn]],
                                ssem[bn]).wait()
                        pltpu.async_copy(w_hbm.at[pl.ds(base + cf * CH, CH)],
                                         w_ring[bn].at[pl.ds(0, CH)], gsem[bn])
                        pltpu.async_copy(dst_hbm.at[wid, cf], dst_ring[bn],
                                         gsem[bn])
                        pltpu.async_copy(h_hbm.at[src_v.at[cf]], rows[bn],
                                         gsem[bn])
            return 0

        lax.fori_loop(0, n_outer, outer, 0)

        for b in range(NBUF):
            pltpu.make_async_copy(rows[b], acc_sh.at[dst_ring[b]],
                                  ssem[b]).wait()

        plsc.subcore_barrier()
        pltpu.sync_copy(acc_sh.at[pl.ds(sid * wb, wb)],
                        out_hbm.at[cid, pl.ds(sid * wb, wb)])

        @pl.when(sid == NS - 1)
        def _wb_tail():
            pltpu.sync_copy(acc_sh.at[pl.ds(NS * wb, rem)],
                            out_hbm.at[cid, pl.ds(NS * wb, rem)])

    return sc_kernel


def kernel(x, edge_index, edge_weight, W1, b1, W2, b2):
    n, d_in = x.shape
    e = edge_weight.shape[0]
    d_h = W1.shape[0]
    d_out = W2.shape[0]

    b1r = b1.reshape(1, d_h)
    b2r = b2.reshape(1, d_out)

    grid = n // ROW_BLK
    h = pl.pallas_call(
        _fc1_body,
        grid=(grid,),
        in_specs=[
            pl.BlockSpec((ROW_BLK, d_in), lambda i: (i, 0)),
            pl.BlockSpec((d_h, d_in), lambda i: (0, 0)),
            pl.BlockSpec((1, d_h), lambda i: (0, 0)),
        ],
        out_specs=pl.BlockSpec((ROW_BLK, d_h), lambda i: (i, 0)),
        out_shape=jax.ShapeDtypeStruct((n, d_h), jnp.float32),
    )(x, W1, b1r)

    nw = NC * NS
    epw = e // nw
    src = edge_index[1].reshape(nw, epw // CH, CH)
    dst = edge_index[0].reshape(nw, epw // CH, CH)
    w = edge_weight
    partials = _make_sc_gather_scatter(n, e, d_h)(h, src, dst, w)

    out = pl.pallas_call(
        _fc2_body,
        grid=(grid,),
        in_specs=[
            pl.BlockSpec((NC, ROW_BLK, d_h), lambda i: (0, i, 0)),
            pl.BlockSpec((d_out, d_h), lambda i: (0, 0)),
            pl.BlockSpec((1, d_out), lambda i: (0, 0)),
        ],
        out_specs=pl.BlockSpec((ROW_BLK, d_out), lambda i: (i, 0)),
        out_shape=jax.ShapeDtypeStruct((n, d_out), jnp.float32),
    )(partials, W2, b2r)
    return out


# speedup vs baseline: 1.1646x; 1.0931x over previous
import functools

import jax
import jax.numpy as jnp
from jax import lax
from jax.experimental import pallas as pl
from jax.experimental.pallas import tpu as pltpu
from jax.experimental.pallas import tpu_sc as plsc

NC = 2
NS = 16
LANES = 16

CH = 40

ROW_BLK = 1000


def _fc1_body(x_ref, w_ref, b_ref, o_ref):
    acc = lax.dot_general(x_ref[...], w_ref[...], (((1,), (1,)), ((), ())),
                          preferred_element_type=jnp.float32)
    o_ref[...] = jnp.maximum(acc + b_ref[...], 0.0)


def _fc2_body(p_ref, w_ref, b_ref, o_ref):
    agg = p_ref[0] + p_ref[1]
    b = b_ref[...]
    g = lax.dot_general(agg, w_ref[...], (((1,), (1,)), ((), ())),
                        preferred_element_type=jnp.float32) + b
    o = lax.dot_general(g, w_ref[...], (((1,), (1,)), ((), ())),
                        preferred_element_type=jnp.float32) + b
    m = jnp.max(o, axis=1, keepdims=True)
    e = jnp.exp(o - m)
    o_ref[...] = e / jnp.sum(e, axis=1, keepdims=True)


NBUF = 3


def _make_sc_gather_scatter(n, e, d):
    assert e % (NC * NS) == 0
    epw = e // (NC * NS)
    assert epw % CH == 0
    n_chunks = epw // CH
    n_outer = (n_chunks + NBUF - 1) // NBUF
    wb = (n // NS) & ~7
    rem = n - NS * wb
    assert rem % 8 == 0

    mesh = plsc.VectorSubcoreMesh(core_axis_name="c", subcore_axis_name="s")

    @functools.partial(
        pl.kernel,
        mesh=mesh,
        out_type=jax.ShapeDtypeStruct((NC, n, d), jnp.float32),
        scratch_types=[
            pltpu.VMEM((n_chunks, CH), jnp.int32),
            [pltpu.VMEM((CH,), jnp.int32) for _ in range(NBUF)],
            [pltpu.VMEM((48,), jnp.float32) for _ in range(NBUF)],
            [pltpu.VMEM((CH, d), jnp.float32) for _ in range(NBUF)],
            pltpu.VMEM_SHARED((n, d), jnp.float32),
            [pltpu.SemaphoreType.DMA for _ in range(NBUF)],
            [pltpu.SemaphoreType.DMA for _ in range(NBUF)],
        ],
    )
    def sc_kernel(h_hbm, src_hbm, dst_hbm, w_hbm, out_hbm,
                  src_v, dst_ring, w_ring, rows, acc_sh, gsem, ssem):
        cid = lax.axis_index("c")
        sid = lax.axis_index("s")
        wid = cid * NS + sid
        base = wid * epw

        pltpu.sync_copy(src_hbm.at[wid], src_v)

        zeros16 = jnp.zeros((LANES,), jnp.float32)

        def zrow(i, _):
            for j in range(d // LANES):
                rows[0][i, pl.ds(j * LANES, LANES)] = zeros16
            return 0

        lax.fori_loop(0, CH, zrow, 0)
        for k in range(wb // CH):
            pltpu.sync_copy(rows[0], acc_sh.at[pl.ds(sid * wb + k * CH, CH)])
        zr = wb - (wb // CH) * CH
        if zr:
            pltpu.sync_copy(rows[0].at[pl.ds(0, zr)],
                            acc_sh.at[pl.ds(sid * wb + (wb // CH) * CH, zr)])

        @pl.when(sid == NS - 1)
        def _zero_tail():
            pltpu.sync_copy(rows[0].at[pl.ds(0, rem)],
                            acc_sh.at[pl.ds(NS * wb, rem)])

        plsc.subcore_barrier()

        for b in range(NBUF - 1):
            pltpu.async_copy(w_hbm.at[pl.ds(base + b * CH, CH)],
                             w_ring[b].at[pl.ds(0, CH)], gsem[b])
            pltpu.async_copy(dst_hbm.at[wid, b], dst_ring[b], gsem[b])
            pltpu.async_copy(h_hbm.at[src_v.at[b]], rows[b], gsem[b])

        def outer(g, _):
            for b in range(NBUF):
                ci = g * NBUF + b

                @pl.when(ci < n_chunks)
                def _step():
                    pltpu.make_async_copy(w_hbm.at[pl.ds(base + ci * CH, CH)],
                                          w_ring[b].at[pl.ds(0, CH)],
                                          gsem[b]).wait()
                    pltpu.make_async_copy(dst_hbm.at[wid, ci], dst_ring[b],
                                          gsem[b]).wait()
                    pltpu.make_async_copy(h_hbm.at[src_v.at[ci]], rows[b],
                                          gsem[b]).wait()

                    def scale16(gg, nrow):
                        w16 = w_ring[b][pl.ds(gg * LANES, LANES)]
                        for i in range(nrow):
                            wv = jnp.full((LANES,), w16[i], dtype=jnp.float32)
                            r = gg * LANES + i
                            for j in range(d // LANES):
                                sl = pl.ds(j * LANES, LANES)
                                rows[b][r, sl] = rows[b][r, sl] * wv

                    bn = (b + NBUF - 1) % NBUF
                    cf = ci + NBUF - 1

                    @pl.when(cf < n_chunks)
                    def _prefetch():
                        @pl.when(ci >= 1)
                        def _drain():
                            pltpu.make_async_copy(
                                rows[bn], acc_sh.at[dst_ring[bn]],
                                ssem[bn]).wait()
                        pltpu.async_copy(w_hbm.at[pl.ds(base + cf * CH, CH)],
                                         w_ring[bn].at[pl.ds(0, CH)], gsem[bn])
                        pltpu.async_copy(dst_hbm.at[wid, cf], dst_ring[bn],
                                         gsem[bn])
                        pltpu.async_copy(h_hbm.at[src_v.at[cf]], rows[bn],
                                         gsem[bn])

                    for gg in range(CH // LANES):
                        scale16(gg, LANES)
                    if CH % LANES:
                        scale16(CH // LANES, CH % LANES)

                    pltpu.async_copy(rows[b], acc_sh.at[dst_ring[b]], ssem[b],
                                     add=True)
            return 0

        lax.fori_loop(0, n_outer, outer, 0)

        for b in range(NBUF):
            pltpu.make_async_copy(rows[b], acc_sh.at[dst_ring[b]],
                                  ssem[b]).wait()

        plsc.subcore_barrier()
        pltpu.sync_copy(acc_sh.at[pl.ds(sid * wb, wb)],
                        out_hbm.at[cid, pl.ds(sid * wb, wb)])

        @pl.when(sid == NS - 1)
        def _wb_tail():
            pltpu.sync_copy(acc_sh.at[pl.ds(NS * wb, rem)],
                            out_hbm.at[cid, pl.ds(NS * wb, rem)])

    return sc_kernel


def kernel(x, edge_index, edge_weight, W1, b1, W2, b2):
    n, d_in = x.shape
    e = edge_weight.shape[0]
    d_h = W1.shape[0]
    d_out = W2.shape[0]

    b1r = b1.reshape(1, d_h)
    b2r = b2.reshape(1, d_out)

    grid = n // ROW_BLK
    h = pl.pallas_call(
        _fc1_body,
        grid=(grid,),
        in_specs=[
            pl.BlockSpec((ROW_BLK, d_in), lambda i: (i, 0)),
            pl.BlockSpec((d_h, d_in), lambda i: (0, 0)),
            pl.BlockSpec((1, d_h), lambda i: (0, 0)),
        ],
        out_specs=pl.BlockSpec((ROW_BLK, d_h), lambda i: (i, 0)),
        out_shape=jax.ShapeDtypeStruct((n, d_h), jnp.float32),
    )(x, W1, b1r)

    nw = NC * NS
    epw = e // nw
    src = edge_index[1].reshape(nw, epw // CH, CH)
    dst = edge_index[0].reshape(nw, epw // CH, CH)
    w = edge_weight
    partials = _make_sc_gather_scatter(n, e, d_h)(h, src, dst, w)

    out = pl.pallas_call(
        _fc2_body,
        grid=(grid,),
        in_specs=[
            pl.BlockSpec((NC, ROW_BLK, d_h), lambda i: (0, i, 0)),
            pl.BlockSpec((d_out, d_h), lambda i: (0, 0)),
            pl.BlockSpec((1, d_out), lambda i: (0, 0)),
        ],
        out_specs=pl.BlockSpec((ROW_BLK, d_out), lambda i: (i, 0)),
        out_shape=jax.ShapeDtypeStruct((n, d_out), jnp.float32),
    )(partials, W2, b2r)
    return out
